# baseline (speedup 1.0000x reference)
import functools

import jax
import jax.numpy as jnp
from jax import lax
from jax.experimental import pallas as pl
from jax.experimental.pallas import tpu as pltpu
from jax.experimental.pallas import tpu_sc as plsc

N = 10000
E = 320000
BD = 128
ND = 128
ID = 256

NC = 2
NS = 16
NW = NC * NS

CHUNK = 128
NCHUNK = E // CHUNK
ROWS_A = 624
ROWS_LAST = N - ROWS_A * (NS - 1)

NSPLIT = 4
EH = E // NSPLIT
HCHUNK = NCHUNK // NSPLIT

TILE = 640
GRIDH = EH // TILE


@functools.lru_cache(maxsize=None)
def _sc_mesh():
    return plsc.VectorSubcoreMesh(
        core_axis_name="c", subcore_axis_name="s",
        num_cores=NC, num_subcores=NS)


def _gather_pair(tabA, idxA, tabB, idxB, h):
    niter = (HCHUNK + NW - 1) // NW

    @functools.partial(
        pl.kernel,
        out_type=(
            jax.ShapeDtypeStruct((EH, BD), jnp.float32),
            jax.ShapeDtypeStruct((EH, BD), jnp.float32),
        ),
        mesh=_sc_mesh(),
        scratch_types=(
            pltpu.VMEM((2, CHUNK), jnp.int32),
            pltpu.VMEM((2, CHUNK), jnp.int32),
            pltpu.VMEM((2, CHUNK, BD), jnp.float32),
            pltpu.VMEM((2, CHUNK, BD), jnp.float32),
            pltpu.SemaphoreType.DMA,
            pltpu.SemaphoreType.DMA,
            pltpu.SemaphoreType.DMA,
        ),
    )
    def k(tA, iA, tB, iB, oA, oB, ia_v, ib_v, ra_v, rb_v, si, sA, sB):
        wid = lax.axis_index("s") * NC + lax.axis_index("c")
        g0 = (h * HCHUNK + wid) * CHUNK

        cp0 = pltpu.async_copy(iA.at[pl.ds(g0, CHUNK)], ia_v.at[0], si)
        cp1 = pltpu.async_copy(iB.at[pl.ds(g0, CHUNK)], ib_v.at[0], si)
        cp0.wait()
        cp1.wait()

        def body(j, carry):
            lc = wid + j * NW
            gbase = (h * HCHUNK + lc) * CHUNK
            obase = lc * CHUNK
            slot = lax.rem(j, 2)
            nslot = 1 - slot

            @pl.when(lc + NW < HCHUNK)
            def _():
                nbase = gbase + NW * CHUNK
                pltpu.async_copy(iA.at[pl.ds(nbase, CHUNK)], ia_v.at[nslot], si)
                pltpu.async_copy(iB.at[pl.ds(nbase, CHUNK)], ib_v.at[nslot], si)

            @pl.when(lc < HCHUNK)
            def _():
                cpA = pltpu.async_copy(tA.at[ia_v.at[slot]], ra_v.at[slot], sA)
                cpB = pltpu.async_copy(tB.at[ib_v.at[slot]], rb_v.at[slot], sB)
                cpA.wait()
                cpB.wait()
                pltpu.sync_copy(ra_v.at[slot], oA.at[pl.ds(obase, CHUNK)])
                pltpu.sync_copy(rb_v.at[slot], oB.at[pl.ds(obase, CHUNK)])

            @pl.when(lc + NW < HCHUNK)
            def _():
                pltpu.make_async_copy(
                    iA.at[pl.ds(0, CHUNK)], ia_v.at[nslot], si).wait()
                pltpu.make_async_copy(
                    iB.at[pl.ds(0, CHUNK)], ib_v.at[nslot], si).wait()

            return carry

        lax.fori_loop(0, niter, body, 0)

    return k(tabA, idxA, tabB, idxB)


def _gather_sum(tabA, idxA, tabB, idxB, h):
    niter = (HCHUNK + NW - 1) // NW

    @functools.partial(
        pl.kernel,
        out_type=jax.ShapeDtypeStruct((EH, BD), jnp.float32),
        mesh=_sc_mesh(),
        scratch_types=(
            pltpu.VMEM((2, CHUNK), jnp.int32),
            pltpu.VMEM((2, CHUNK), jnp.int32),
            pltpu.VMEM((2, CHUNK, BD), jnp.float32),
            pltpu.VMEM((2, CHUNK, BD), jnp.float32),
            pltpu.SemaphoreType.DMA,
            pltpu.SemaphoreType.DMA,
            pltpu.SemaphoreType.DMA,
        ),
    )
    def k(tA, iA, tB, iB, oS, ia_v, ib_v, ra_v, rb_v, si, sA, sB):
        wid = lax.axis_index("s") * NC + lax.axis_index("c")
        g0 = (h * HCHUNK + wid) * CHUNK

        cp0 = pltpu.async_copy(iA.at[pl.ds(g0, CHUNK)], ia_v.at[0], si)
        cp1 = pltpu.async_copy(iB.at[pl.ds(g0, CHUNK)], ib_v.at[0], si)
        cp0.wait()
        cp1.wait()

        def body(j, carry):
            lc = wid + j * NW
            gbase = (h * HCHUNK + lc) * CHUNK
            obase = lc * CHUNK
            slot = lax.rem(j, 2)
            nslot = 1 - slot

            @pl.when(lc + NW < HCHUNK)
            def _():
                nbase = gbase + NW * CHUNK
                pltpu.async_copy(iA.at[pl.ds(nbase, CHUNK)], ia_v.at[nslot], si)
                pltpu.async_copy(iB.at[pl.ds(nbase, CHUNK)], ib_v.at[nslot], si)

            @pl.when(lc < HCHUNK)
            def _():
                cpA = pltpu.async_copy(tA.at[ia_v.at[slot]], ra_v.at[slot], sA)
                cpB = pltpu.async_copy(tB.at[ib_v.at[slot]], rb_v.at[slot], sB)
                cpA.wait()
                cpB.wait()

                def add_row(r, carry2):
                    for cb in range(BD // 16):
                        sl = pl.ds(cb * 16, 16)
                        ra_v[slot, r, sl] = ra_v[slot, r, sl] + rb_v[
                            slot, r, sl]
                    return carry2

                lax.fori_loop(0, CHUNK, add_row, 0)
                pltpu.sync_copy(ra_v.at[slot], oS.at[pl.ds(obase, CHUNK)])

            @pl.when(lc + NW < HCHUNK)
            def _():
                pltpu.make_async_copy(
                    iA.at[pl.ds(0, CHUNK)], ia_v.at[nslot], si).wait()
                pltpu.make_async_copy(
                    iB.at[pl.ds(0, CHUNK)], ib_v.at[nslot], si).wait()

            return carry

        lax.fori_loop(0, niter, body, 0)

    return k(tabA, idxA, tabB, idxB)


def _scatter_pair(mL, idx_r, mR, idx_l, zeros_n, h):
    niter = (HCHUNK + NS - 1) // NS

    @functools.partial(
        pl.kernel,
        out_type=(
            jax.ShapeDtypeStruct((N, BD), jnp.float32),
            jax.ShapeDtypeStruct((N, BD), jnp.float32),
        ),
        mesh=_sc_mesh(),
        scratch_types=(
            pltpu.VMEM((2, CHUNK), jnp.int32),
            pltpu.VMEM((2, CHUNK, BD), jnp.float32),
            pltpu.VMEM_SHARED((N, BD), jnp.float32),
            pltpu.SemaphoreType.DMA,
        ),
    )
    def k(mLr, iR, mRr, iL, zr, oL, oR, idx_v, rows_v, acc, sin):
        cid = lax.axis_index("c")
        sid = lax.axis_index("s")
        r0 = pl.multiple_of(sid * ROWS_A, 8)

        @pl.when(sid < NS - 1)
        def _():
            pltpu.sync_copy(zr.at[pl.ds(r0, ROWS_A)], acc.at[pl.ds(r0, ROWS_A)])

        @pl.when(sid == NS - 1)
        def _():
            pltpu.sync_copy(
                zr.at[pl.ds(r0, ROWS_LAST)], acc.at[pl.ds(r0, ROWS_LAST)]
            )

        plsc.subcore_barrier()

        lb0 = sid * CHUNK
        gb0 = h * HCHUNK * CHUNK + lb0

        @pl.when(cid == 0)
        def _():
            pltpu.sync_copy(iR.at[pl.ds(gb0, CHUNK)], idx_v.at[0])
            pltpu.sync_copy(mLr.at[pl.ds(lb0, CHUNK)], rows_v.at[0])

        @pl.when(cid == 1)
        def _():
            pltpu.sync_copy(iL.at[pl.ds(gb0, CHUNK)], idx_v.at[0])
            pltpu.sync_copy(mRr.at[pl.ds(lb0, CHUNK)], rows_v.at[0])

        def body(j, carry):
            lc = sid + j * NS
            slot = lax.rem(j, 2)
            nslot = 1 - slot
            nlbase = (lc + NS) * CHUNK
            ngbase = h * HCHUNK * CHUNK + nlbase

            @pl.when(jnp.logical_and(cid == 0, lc + NS < HCHUNK))
            def _():
                pltpu.async_copy(iR.at[pl.ds(ngbase, CHUNK)],
                                 idx_v.at[nslot], sin)
                pltpu.async_copy(mLr.at[pl.ds(nlbase, CHUNK)],
                                 rows_v.at[nslot], sin)

            @pl.when(jnp.logical_and(cid == 1, lc + NS < HCHUNK))
            def _():
                pltpu.async_copy(iL.at[pl.ds(ngbase, CHUNK)],
                                 idx_v.at[nslot], sin)
                pltpu.async_copy(mRr.at[pl.ds(nlbase, CHUNK)],
                                 rows_v.at[nslot], sin)

            @pl.when(lc < HCHUNK)
            def _():
                pltpu.sync_copy(rows_v.at[slot], acc.at[idx_v.at[slot]],
                                add=True)

            @pl.when(lc + NS < HCHUNK)
            def _():
                pltpu.make_async_copy(
                    iR.at[pl.ds(0, CHUNK)], idx_v.at[nslot], sin).wait()
                pltpu.make_async_copy(
                    mLr.at[pl.ds(0, CHUNK)], rows_v.at[nslot], sin).wait()

            return carry

        lax.fori_loop(0, niter, body, 0)
        plsc.subcore_barrier()

        @pl.when(jnp.logical_and(cid == 0, sid < NS - 1))
        def _():
            pltpu.sync_copy(acc.at[pl.ds(r0, ROWS_A)], oL.at[pl.ds(r0, ROWS_A)])

        @pl.when(jnp.logical_and(cid == 0, sid == NS - 1))
        def _():
            pltpu.sync_copy(
                acc.at[pl.ds(r0, ROWS_LAST)], oL.at[pl.ds(r0, ROWS_LAST)]
            )

        @pl.when(jnp.logical_and(cid == 1, sid < NS - 1))
        def _():
            pltpu.sync_copy(acc.at[pl.ds(r0, ROWS_A)], oR.at[pl.ds(r0, ROWS_A)])

        @pl.when(jnp.logical_and(cid == 1, sid == NS - 1))
        def _():
            pltpu.sync_copy(
                acc.at[pl.ds(r0, ROWS_LAST)], oR.at[pl.ds(r0, ROWS_LAST)]
            )

    return k(mL, idx_r, mR, idx_l, zeros_n)


def _ffn_body(hb_ref, hl_ref, hr_ref, wbl, wnl, w1l, b1l, w2l, b2l, wbr, wnr,
              w1r, b1r, w2r, b2r, wnlt, wnrt, wst, bsk, ml_ref, mr_ref,
              sk_ref):
    f32 = jnp.float32
    bf = jnp.bfloat16
    hb = hb_ref[...].astype(bf)
    hl = hl_ref[...].astype(bf)
    hr = hr_ref[...].astype(bf)

    interL = (jnp.dot(hb, wbl[...], preferred_element_type=f32) * jnp.dot(
        hl, wnl[...], preferred_element_type=f32))
    aL = jnp.maximum(
        jnp.dot(interL.astype(bf), w1l[...], preferred_element_type=f32)
        + b1l[...], 0.0)
    ml_ref[...] = jnp.dot(
        aL.astype(bf), w2l[...], preferred_element_type=f32) + b2l[...]

    interR = (jnp.dot(hb, wbr[...], preferred_element_type=f32) * jnp.dot(
        hr, wnr[...], preferred_element_type=f32))
    aR = jnp.maximum(
        jnp.dot(interR.astype(bf), w1r[...], preferred_element_type=f32)
        + b1r[...], 0.0)
    mr_ref[...] = jnp.dot(
        aR.astype(bf), w2r[...], preferred_element_type=f32) + b2r[...]

    sk_ref[...] = (
        jnp.dot(hl, wnlt[...], preferred_element_type=f32)
        + jnp.dot(hr, wnrt[...], preferred_element_type=f32)
        + jnp.dot(hb, wst[...], preferred_element_type=f32)
        + bsk[...]
    ).astype(bf)


def _final_body(po_ref, ms_ref, sk_ref, ln_g, ln_b, wot, bo, out_ref):
    del po_ref
    x = ms_ref[...] + sk_ref[...].astype(jnp.float32)
    m = jnp.mean(x, axis=-1, keepdims=True)
    xc = x - m
    v = jnp.mean(xc * xc, axis=-1, keepdims=True)
    xn = xc * lax.rsqrt(v + 1e-5) * ln_g[...] + ln_b[...]
    out_ref[...] = (
        jnp.dot(jnp.maximum(xn, 0.0), wot[...],
                preferred_element_type=jnp.float32)
        + bo[...]
    )


def _slice_spec(h):
    return pl.BlockSpec((TILE, BD), lambda i: (i + h * GRIDH, 0))


def _local_spec():
    return pl.BlockSpec((TILE, BD), lambda i: (i, 0))


def _w_spec(r, c):
    return pl.BlockSpec((r, c), lambda i: (0, 0))


def kernel(h_bond, bond_index, h_node, fL_Wb, fL_Wn, fL_W1, fL_b1, fL_W2,
           fL_b2, fR_Wb, fR_Wn, fR_W1, fR_b1, fR_W2, fR_b2, Wnl, bnl, Wnr,
           bnr, Ws, bs, ln_g, ln_b, Wo, bo):
    left = bond_index[0]
    right = bond_index[1]
    bf = jnp.bfloat16
    f32 = jnp.float32

    bsk = (bnl + bnr + bs).reshape(1, BD)
    ffn_w = (
        fL_Wb.T.astype(bf), fL_Wn.T.astype(bf),
        fL_W1.T.astype(bf), fL_b1.reshape(1, ID),
        fL_W2.T.astype(bf), fL_b2.reshape(1, BD),
        fR_Wb.T.astype(bf), fR_Wn.T.astype(bf),
        fR_W1.T.astype(bf), fR_b1.reshape(1, ID),
        fR_W2.T.astype(bf), fR_b2.reshape(1, BD),
        Wnl.T.astype(bf), Wnr.T.astype(bf), Ws.T.astype(bf), bsk,
    )
    ffn_w_specs = [
        _w_spec(BD, ID), _w_spec(ND, ID), _w_spec(ID, ID), _w_spec(1, ID),
        _w_spec(ID, BD), _w_spec(1, BD),
        _w_spec(BD, ID), _w_spec(ND, ID), _w_spec(ID, ID), _w_spec(1, ID),
        _w_spec(ID, BD), _w_spec(1, BD),
        _w_spec(ND, BD), _w_spec(ND, BD), _w_spec(BD, BD), _w_spec(1, BD),
    ]
    zeros_n = jnp.zeros((N, BD), f32)

    mLs, mRs, skips, sLs, sRs = [], [], [], [], []
    for h in range(NSPLIT):
        hnL, hnR = _gather_pair(h_node, left, h_node, right, h)
        mL, mR, skip = pl.pallas_call(
            _ffn_body,
            grid=(GRIDH,),
            in_specs=[_slice_spec(h), _local_spec(), _local_spec()]
            + ffn_w_specs,
            out_specs=[_local_spec(), _local_spec(), _local_spec()],
            out_shape=[
                jax.ShapeDtypeStruct((EH, BD), f32),
                jax.ShapeDtypeStruct((EH, BD), f32),
                jax.ShapeDtypeStruct((EH, BD), bf),
            ],
        )(h_bond, hnL, hnR, *ffn_w)
        sL_h, sR_h = _scatter_pair(mL, right, mR, left, zeros_n, h)
        mLs.append(mL)
        mRs.append(mR)
        skips.append(skip)
        sLs.append(sL_h)
        sRs.append(sR_h)

    sL = sLs[0]
    sR = sRs[0]
    for h in range(1, NSPLIT):
        sL = sL + sLs[h]
        sR = sR + sRs[h]

    out = None
    fin_w = (ln_g.reshape(1, BD), ln_b.reshape(1, BD), Wo.T,
             bo.reshape(1, BD))
    fin_w_specs = [_w_spec(1, BD), _w_spec(1, BD), _w_spec(BD, BD),
                   _w_spec(1, BD)]
    for h in range(NSPLIT):
        mSg = _gather_sum(sL, left, sR, right, h)
        if out is None:
            prev = jnp.zeros((8, BD), f32)
            aliases = {}
        else:
            prev = out
            aliases = {0: 0}
        out = pl.pallas_call(
            _final_body,
            grid=(GRIDH,),
            in_specs=[pl.BlockSpec(memory_space=pltpu.MemorySpace.HBM),
                      _local_spec(), _local_spec()]
            + fin_w_specs,
            out_specs=pl.BlockSpec((TILE, BD), lambda i, h=h: (i + h * GRIDH,
                                                               0)),
            out_shape=jax.ShapeDtypeStruct((E, BD), f32),
            input_output_aliases=aliases,
        )(prev, mSg, skips[h], *fin_w)
    return out


# speedup vs baseline: 1.2819x; 1.2819x over previous
import functools

import jax
import jax.numpy as jnp
from jax import lax
from jax.experimental import pallas as pl
from jax.experimental.pallas import tpu as pltpu
from jax.experimental.pallas import tpu_sc as plsc

N = 10000
E = 320000
BD = 128
ND = 128
ID = 256

NC = 2
NS = 16
NW = NC * NS

CHUNK = 128
NCHUNK = E // CHUNK
ROWS_A = 624
ROWS_LAST = N - ROWS_A * (NS - 1)

NSPLIT = 4
EH = E // NSPLIT
HCHUNK = NCHUNK // NSPLIT

TILE = 1000
GRIDH = EH // TILE


@functools.lru_cache(maxsize=None)
def _sc_mesh():
    return plsc.VectorSubcoreMesh(
        core_axis_name="c", subcore_axis_name="s",
        num_cores=NC, num_subcores=NS)


def _gather_pair(tabA, idxA, tabB, idxB, h):
    niter = (HCHUNK + NW - 1) // NW

    @functools.partial(
        pl.kernel,
        out_type=(
            jax.ShapeDtypeStruct((EH, BD), jnp.float32),
            jax.ShapeDtypeStruct((EH, BD), jnp.float32),
        ),
        mesh=_sc_mesh(),
        scratch_types=(
            pltpu.VMEM((2, CHUNK), jnp.int32),
            pltpu.VMEM((2, CHUNK), jnp.int32),
            pltpu.VMEM((2, CHUNK, BD), jnp.float32),
            pltpu.VMEM((2, CHUNK, BD), jnp.float32),
            pltpu.SemaphoreType.DMA,
            pltpu.SemaphoreType.DMA,
            pltpu.SemaphoreType.DMA,
        ),
    )
    def k(tA, iA, tB, iB, oA, oB, ia_v, ib_v, ra_v, rb_v, si, sA, sB):
        wid = lax.axis_index("s") * NC + lax.axis_index("c")
        g0 = (h * HCHUNK + wid) * CHUNK

        cp0 = pltpu.async_copy(iA.at[pl.ds(g0, CHUNK)], ia_v.at[0], si)
        cp1 = pltpu.async_copy(iB.at[pl.ds(g0, CHUNK)], ib_v.at[0], si)
        cp0.wait()
        cp1.wait()

        def body(j, carry):
            lc = wid + j * NW
            gbase = (h * HCHUNK + lc) * CHUNK
            obase = lc * CHUNK
            slot = lax.rem(j, 2)
            nslot = 1 - slot

            @pl.when(lc + NW < HCHUNK)
            def _():
                nbase = gbase + NW * CHUNK
                pltpu.async_copy(iA.at[pl.ds(nbase, CHUNK)], ia_v.at[nslot], si)
                pltpu.async_copy(iB.at[pl.ds(nbase, CHUNK)], ib_v.at[nslot], si)

            @pl.when(lc < HCHUNK)
            def _():
                cpA = pltpu.async_copy(tA.at[ia_v.at[slot]], ra_v.at[slot], sA)
                cpB = pltpu.async_copy(tB.at[ib_v.at[slot]], rb_v.at[slot], sB)
                cpA.wait()
                cpB.wait()
                pltpu.sync_copy(ra_v.at[slot], oA.at[pl.ds(obase, CHUNK)])
                pltpu.sync_copy(rb_v.at[slot], oB.at[pl.ds(obase, CHUNK)])

            @pl.when(lc + NW < HCHUNK)
            def _():
                pltpu.make_async_copy(
                    iA.at[pl.ds(0, CHUNK)], ia_v.at[nslot], si).wait()
                pltpu.make_async_copy(
                    iB.at[pl.ds(0, CHUNK)], ib_v.at[nslot], si).wait()

            return carry

        lax.fori_loop(0, niter, body, 0)

    return k(tabA, idxA, tabB, idxB)


def _scatter_pair(mL, idx_r, mR, idx_l, zeros_n, h):
    niter = (HCHUNK + NS - 1) // NS

    @functools.partial(
        pl.kernel,
        out_type=(
            jax.ShapeDtypeStruct((N, BD), jnp.float32),
            jax.ShapeDtypeStruct((N, BD), jnp.float32),
        ),
        mesh=_sc_mesh(),
        scratch_types=(
            pltpu.VMEM((2, CHUNK), jnp.int32),
            pltpu.VMEM((2, CHUNK, BD), jnp.float32),
            pltpu.VMEM_SHARED((N, BD), jnp.float32),
            pltpu.SemaphoreType.DMA,
        ),
    )
    def k(mLr, iR, mRr, iL, zr, oL, oR, idx_v, rows_v, acc, sin):
        cid = lax.axis_index("c")
        sid = lax.axis_index("s")
        r0 = pl.multiple_of(sid * ROWS_A, 8)

        @pl.when(sid < NS - 1)
        def _():
            pltpu.sync_copy(zr.at[pl.ds(r0, ROWS_A)], acc.at[pl.ds(r0, ROWS_A)])

        @pl.when(sid == NS - 1)
        def _():
            pltpu.sync_copy(
                zr.at[pl.ds(r0, ROWS_LAST)], acc.at[pl.ds(r0, ROWS_LAST)]
            )

        plsc.subcore_barrier()

        lb0 = sid * CHUNK
        gb0 = h * HCHUNK * CHUNK + lb0

        @pl.when(cid == 0)
        def _():
            pltpu.sync_copy(iR.at[pl.ds(gb0, CHUNK)], idx_v.at[0])
            pltpu.sync_copy(mLr.at[pl.ds(lb0, CHUNK)], rows_v.at[0])

        @pl.when(cid == 1)
        def _():
            pltpu.sync_copy(iL.at[pl.ds(gb0, CHUNK)], idx_v.at[0])
            pltpu.sync_copy(mRr.at[pl.ds(lb0, CHUNK)], rows_v.at[0])

        def body(j, carry):
            lc = sid + j * NS
            slot = lax.rem(j, 2)
            nslot = 1 - slot
            nlbase = (lc + NS) * CHUNK
            ngbase = h * HCHUNK * CHUNK + nlbase

            @pl.when(jnp.logical_and(cid == 0, lc + NS < HCHUNK))
            def _():
                pltpu.async_copy(iR.at[pl.ds(ngbase, CHUNK)],
                                 idx_v.at[nslot], sin)
                pltpu.async_copy(mLr.at[pl.ds(nlbase, CHUNK)],
                                 rows_v.at[nslot], sin)

            @pl.when(jnp.logical_and(cid == 1, lc + NS < HCHUNK))
            def _():
                pltpu.async_copy(iL.at[pl.ds(ngbase, CHUNK)],
                                 idx_v.at[nslot], sin)
                pltpu.async_copy(mRr.at[pl.ds(nlbase, CHUNK)],
                                 rows_v.at[nslot], sin)

            @pl.when(lc < HCHUNK)
            def _():
                pltpu.sync_copy(rows_v.at[slot], acc.at[idx_v.at[slot]],
                                add=True)

            @pl.when(lc + NS < HCHUNK)
            def _():
                pltpu.make_async_copy(
                    iR.at[pl.ds(0, CHUNK)], idx_v.at[nslot], sin).wait()
                pltpu.make_async_copy(
                    mLr.at[pl.ds(0, CHUNK)], rows_v.at[nslot], sin).wait()

            return carry

        lax.fori_loop(0, niter, body, 0)
        plsc.subcore_barrier()

        @pl.when(jnp.logical_and(cid == 0, sid < NS - 1))
        def _():
            pltpu.sync_copy(acc.at[pl.ds(r0, ROWS_A)], oL.at[pl.ds(r0, ROWS_A)])

        @pl.when(jnp.logical_and(cid == 0, sid == NS - 1))
        def _():
            pltpu.sync_copy(
                acc.at[pl.ds(r0, ROWS_LAST)], oL.at[pl.ds(r0, ROWS_LAST)]
            )

        @pl.when(jnp.logical_and(cid == 1, sid < NS - 1))
        def _():
            pltpu.sync_copy(acc.at[pl.ds(r0, ROWS_A)], oR.at[pl.ds(r0, ROWS_A)])

        @pl.when(jnp.logical_and(cid == 1, sid == NS - 1))
        def _():
            pltpu.sync_copy(
                acc.at[pl.ds(r0, ROWS_LAST)], oR.at[pl.ds(r0, ROWS_LAST)]
            )

    return k(mL, idx_r, mR, idx_l, zeros_n)


def _ffn_body(hb_ref, hl_ref, hr_ref, wbl, wnl, w1l, b1l, w2l, b2l, wbr, wnr,
              w1r, b1r, w2r, b2r, wnlt, wnrt, wst, bsk, ml_ref, mr_ref,
              sk_ref):
    f32 = jnp.float32
    bf = jnp.bfloat16
    hb = hb_ref[...].astype(bf)
    hl = hl_ref[...].astype(bf)
    hr = hr_ref[...].astype(bf)

    interL = (jnp.dot(hb, wbl[...], preferred_element_type=f32) * jnp.dot(
        hl, wnl[...], preferred_element_type=f32))
    aL = jnp.maximum(
        jnp.dot(interL.astype(bf), w1l[...], preferred_element_type=f32)
        + b1l[...], 0.0)
    ml_ref[...] = jnp.dot(
        aL.astype(bf), w2l[...], preferred_element_type=f32) + b2l[...]

    interR = (jnp.dot(hb, wbr[...], preferred_element_type=f32) * jnp.dot(
        hr, wnr[...], preferred_element_type=f32))
    aR = jnp.maximum(
        jnp.dot(interR.astype(bf), w1r[...], preferred_element_type=f32)
        + b1r[...], 0.0)
    mr_ref[...] = jnp.dot(
        aR.astype(bf), w2r[...], preferred_element_type=f32) + b2r[...]

    sk_ref[...] = (
        jnp.dot(hl, wnlt[...], preferred_element_type=f32)
        + jnp.dot(hr, wnrt[...], preferred_element_type=f32)
        + jnp.dot(hb, wst[...], preferred_element_type=f32)
        + bsk[...]
    ).astype(bf)


def _final_body(po_ref, ml_ref, mr_ref, sk_ref, ln_g, ln_b, wot, bo, out_ref):
    del po_ref
    x = ml_ref[...] + mr_ref[...] + sk_ref[...].astype(jnp.float32)
    m = jnp.mean(x, axis=-1, keepdims=True)
    xc = x - m
    v = jnp.mean(xc * xc, axis=-1, keepdims=True)
    xn = xc * lax.rsqrt(v + 1e-5) * ln_g[...] + ln_b[...]
    out_ref[...] = (
        jnp.dot(jnp.maximum(xn, 0.0), wot[...],
                preferred_element_type=jnp.float32)
        + bo[...]
    )


def _slice_spec(h):
    return pl.BlockSpec((TILE, BD), lambda i: (i + h * GRIDH, 0))


def _local_spec():
    return pl.BlockSpec((TILE, BD), lambda i: (i, 0))


def _w_spec(r, c):
    return pl.BlockSpec((r, c), lambda i: (0, 0))


def kernel(h_bond, bond_index, h_node, fL_Wb, fL_Wn, fL_W1, fL_b1, fL_W2,
           fL_b2, fR_Wb, fR_Wn, fR_W1, fR_b1, fR_W2, fR_b2, Wnl, bnl, Wnr,
           bnr, Ws, bs, ln_g, ln_b, Wo, bo):
    left = bond_index[0]
    right = bond_index[1]
    bf = jnp.bfloat16
    f32 = jnp.float32

    bsk = (bnl + bnr + bs).reshape(1, BD)
    ffn_w = (
        fL_Wb.T.astype(bf), fL_Wn.T.astype(bf),
        fL_W1.T.astype(bf), fL_b1.reshape(1, ID),
        fL_W2.T.astype(bf), fL_b2.reshape(1, BD),
        fR_Wb.T.astype(bf), fR_Wn.T.astype(bf),
        fR_W1.T.astype(bf), fR_b1.reshape(1, ID),
        fR_W2.T.astype(bf), fR_b2.reshape(1, BD),
        Wnl.T.astype(bf), Wnr.T.astype(bf), Ws.T.astype(bf), bsk,
    )
    ffn_w_specs = [
        _w_spec(BD, ID), _w_spec(ND, ID), _w_spec(ID, ID), _w_spec(1, ID),
        _w_spec(ID, BD), _w_spec(1, BD),
        _w_spec(BD, ID), _w_spec(ND, ID), _w_spec(ID, ID), _w_spec(1, ID),
        _w_spec(ID, BD), _w_spec(1, BD),
        _w_spec(ND, BD), _w_spec(ND, BD), _w_spec(BD, BD), _w_spec(1, BD),
    ]
    zeros_n = jnp.zeros((N, BD), f32)

    mLs, mRs, skips, sLs, sRs = [], [], [], [], []
    for h in range(NSPLIT):
        hnL, hnR = _gather_pair(h_node, left, h_node, right, h)
        mL, mR, skip = pl.pallas_call(
            _ffn_body,
            grid=(GRIDH,),
            in_specs=[_slice_spec(h), _local_spec(), _local_spec()]
            + ffn_w_specs,
            out_specs=[_local_spec(), _local_spec(), _local_spec()],
            out_shape=[
                jax.ShapeDtypeStruct((EH, BD), f32),
                jax.ShapeDtypeStruct((EH, BD), f32),
                jax.ShapeDtypeStruct((EH, BD), bf),
            ],
        )(h_bond, hnL, hnR, *ffn_w)
        sL_h, sR_h = _scatter_pair(mL, right, mR, left, zeros_n, h)
        mLs.append(mL)
        mRs.append(mR)
        skips.append(skip)
        sLs.append(sL_h)
        sRs.append(sR_h)

    sL = sLs[0]
    sR = sRs[0]
    for h in range(1, NSPLIT):
        sL = sL + sLs[h]
        sR = sR + sRs[h]

    out = None
    fin_w = (ln_g.reshape(1, BD), ln_b.reshape(1, BD), Wo.T,
             bo.reshape(1, BD))
    fin_w_specs = [_w_spec(1, BD), _w_spec(1, BD), _w_spec(BD, BD),
                   _w_spec(1, BD)]
    for h in range(NSPLIT):
        mLg, mRg = _gather_pair(sL, left, sR, right, h)
        if out is None:
            prev = jnp.zeros((8, BD), f32)
            aliases = {}
        else:
            prev = out
            aliases = {0: 0}
        out = pl.pallas_call(
            _final_body,
            grid=(GRIDH,),
            in_specs=[pl.BlockSpec(memory_space=pltpu.MemorySpace.HBM),
                      _local_spec(), _local_spec(), _local_spec()]
            + fin_w_specs,
            out_specs=pl.BlockSpec((TILE, BD), lambda i, h=h: (i + h * GRIDH,
                                                               0)),
            out_shape=jax.ShapeDtypeStruct((E, BD), f32),
            input_output_aliases=aliases,
        )(prev, mLg, mRg, skips[h], *fin_w)
    return out


# speedup vs baseline: 1.4385x; 1.1221x over previous
import functools

import jax
import jax.numpy as jnp
from jax import lax
from jax.experimental import pallas as pl
from jax.experimental.pallas import tpu as pltpu
from jax.experimental.pallas import tpu_sc as plsc

N = 10000
E = 320000
BD = 128
ND = 128
ID = 256

NC = 2
NS = 16
NW = NC * NS

CHUNK = 128
NCHUNK = E // CHUNK
ROWS_A = 624
ROWS_LAST = N - ROWS_A * (NS - 1)

NSPLIT = 4
EH = E // NSPLIT
HCHUNK = NCHUNK // NSPLIT

TILE = 2000
GRIDH = EH // TILE


@functools.lru_cache(maxsize=None)
def _sc_mesh():
    return plsc.VectorSubcoreMesh(
        core_axis_name="c", subcore_axis_name="s",
        num_cores=NC, num_subcores=NS)


def _gather_pair(tabA, idxA, tabB, idxB, h):
    niter = (HCHUNK + NW - 1) // NW

    @functools.partial(
        pl.kernel,
        out_type=(
            jax.ShapeDtypeStruct((EH, BD), jnp.float32),
            jax.ShapeDtypeStruct((EH, BD), jnp.float32),
        ),
        mesh=_sc_mesh(),
        scratch_types=(
            pltpu.VMEM((2, CHUNK), jnp.int32),
            pltpu.VMEM((2, CHUNK), jnp.int32),
            pltpu.VMEM((2, CHUNK, BD), jnp.float32),
            pltpu.VMEM((2, CHUNK, BD), jnp.float32),
            pltpu.SemaphoreType.DMA,
            pltpu.SemaphoreType.DMA,
            pltpu.SemaphoreType.DMA,
        ),
    )
    def k(tA, iA, tB, iB, oA, oB, ia_v, ib_v, ra_v, rb_v, si, sA, sB):
        wid = lax.axis_index("s") * NC + lax.axis_index("c")
        g0 = (h * HCHUNK + wid) * CHUNK

        cp0 = pltpu.async_copy(iA.at[pl.ds(g0, CHUNK)], ia_v.at[0], si)
        cp1 = pltpu.async_copy(iB.at[pl.ds(g0, CHUNK)], ib_v.at[0], si)
        cp0.wait()
        cp1.wait()

        def body(j, carry):
            lc = wid + j * NW
            gbase = (h * HCHUNK + lc) * CHUNK
            obase = lc * CHUNK
            slot = lax.rem(j, 2)
            nslot = 1 - slot

            @pl.when(lc + NW < HCHUNK)
            def _():
                nbase = gbase + NW * CHUNK
                pltpu.async_copy(iA.at[pl.ds(nbase, CHUNK)], ia_v.at[nslot], si)
                pltpu.async_copy(iB.at[pl.ds(nbase, CHUNK)], ib_v.at[nslot], si)

            @pl.when(lc < HCHUNK)
            def _():
                cpA = pltpu.async_copy(tA.at[ia_v.at[slot]], ra_v.at[slot], sA)
                cpB = pltpu.async_copy(tB.at[ib_v.at[slot]], rb_v.at[slot], sB)
                cpA.wait()
                cpB.wait()
                pltpu.sync_copy(ra_v.at[slot], oA.at[pl.ds(obase, CHUNK)])
                pltpu.sync_copy(rb_v.at[slot], oB.at[pl.ds(obase, CHUNK)])

            @pl.when(lc + NW < HCHUNK)
            def _():
                pltpu.make_async_copy(
                    iA.at[pl.ds(0, CHUNK)], ia_v.at[nslot], si).wait()
                pltpu.make_async_copy(
                    iB.at[pl.ds(0, CHUNK)], ib_v.at[nslot], si).wait()

            return carry

        lax.fori_loop(0, niter, body, 0)

    return k(tabA, idxA, tabB, idxB)


def _scatter_pair(mL, idx_r, mR, idx_l, zeros_n, h):
    niter = (HCHUNK + NS - 1) // NS

    @functools.partial(
        pl.kernel,
        out_type=(
            jax.ShapeDtypeStruct((N, BD), jnp.float32),
            jax.ShapeDtypeStruct((N, BD), jnp.float32),
        ),
        mesh=_sc_mesh(),
        scratch_types=(
            pltpu.VMEM((2, CHUNK), jnp.int32),
            pltpu.VMEM((2, CHUNK, BD), jnp.float32),
            pltpu.VMEM_SHARED((N, BD), jnp.float32),
            pltpu.SemaphoreType.DMA,
        ),
    )
    def k(mLr, iR, mRr, iL, zr, oL, oR, idx_v, rows_v, acc, sin):
        cid = lax.axis_index("c")
        sid = lax.axis_index("s")
        r0 = pl.multiple_of(sid * ROWS_A, 8)

        @pl.when(sid < NS - 1)
        def _():
            pltpu.sync_copy(zr.at[pl.ds(r0, ROWS_A)], acc.at[pl.ds(r0, ROWS_A)])

        @pl.when(sid == NS - 1)
        def _():
            pltpu.sync_copy(
                zr.at[pl.ds(r0, ROWS_LAST)], acc.at[pl.ds(r0, ROWS_LAST)]
            )

        plsc.subcore_barrier()

        lb0 = sid * CHUNK
        gb0 = h * HCHUNK * CHUNK + lb0

        @pl.when(cid == 0)
        def _():
            pltpu.sync_copy(iR.at[pl.ds(gb0, CHUNK)], idx_v.at[0])
            pltpu.sync_copy(mLr.at[pl.ds(lb0, CHUNK)], rows_v.at[0])

        @pl.when(cid == 1)
        def _():
            pltpu.sync_copy(iL.at[pl.ds(gb0, CHUNK)], idx_v.at[0])
            pltpu.sync_copy(mRr.at[pl.ds(lb0, CHUNK)], rows_v.at[0])

        def body(j, carry):
            lc = sid + j * NS
            slot = lax.rem(j, 2)
            nslot = 1 - slot
            nlbase = (lc + NS) * CHUNK
            ngbase = h * HCHUNK * CHUNK + nlbase

            @pl.when(jnp.logical_and(cid == 0, lc + NS < HCHUNK))
            def _():
                pltpu.async_copy(iR.at[pl.ds(ngbase, CHUNK)],
                                 idx_v.at[nslot], sin)
                pltpu.async_copy(mLr.at[pl.ds(nlbase, CHUNK)],
                                 rows_v.at[nslot], sin)

            @pl.when(jnp.logical_and(cid == 1, lc + NS < HCHUNK))
            def _():
                pltpu.async_copy(iL.at[pl.ds(ngbase, CHUNK)],
                                 idx_v.at[nslot], sin)
                pltpu.async_copy(mRr.at[pl.ds(nlbase, CHUNK)],
                                 rows_v.at[nslot], sin)

            @pl.when(lc < HCHUNK)
            def _():
                pltpu.sync_copy(rows_v.at[slot], acc.at[idx_v.at[slot]],
                                add=True)

            @pl.when(lc + NS < HCHUNK)
            def _():
                pltpu.make_async_copy(
                    iR.at[pl.ds(0, CHUNK)], idx_v.at[nslot], sin).wait()
                pltpu.make_async_copy(
                    mLr.at[pl.ds(0, CHUNK)], rows_v.at[nslot], sin).wait()

            return carry

        lax.fori_loop(0, niter, body, 0)
        plsc.subcore_barrier()

        @pl.when(jnp.logical_and(cid == 0, sid < NS - 1))
        def _():
            pltpu.sync_copy(acc.at[pl.ds(r0, ROWS_A)], oL.at[pl.ds(r0, ROWS_A)])

        @pl.when(jnp.logical_and(cid == 0, sid == NS - 1))
        def _():
            pltpu.sync_copy(
                acc.at[pl.ds(r0, ROWS_LAST)], oL.at[pl.ds(r0, ROWS_LAST)]
            )

        @pl.when(jnp.logical_and(cid == 1, sid < NS - 1))
        def _():
            pltpu.sync_copy(acc.at[pl.ds(r0, ROWS_A)], oR.at[pl.ds(r0, ROWS_A)])

        @pl.when(jnp.logical_and(cid == 1, sid == NS - 1))
        def _():
            pltpu.sync_copy(
                acc.at[pl.ds(r0, ROWS_LAST)], oR.at[pl.ds(r0, ROWS_LAST)]
            )

    return k(mL, idx_r, mR, idx_l, zeros_n)


def _ffn_body(hb_ref, hl_ref, hr_ref, wbl, wnl, w1l, b1l, w2l, b2l, wbr, wnr,
              w1r, b1r, w2r, b2r, wnlt, wnrt, wst, bsk, ml_ref, mr_ref,
              sk_ref):
    f32 = jnp.float32
    bf = jnp.bfloat16
    hb = hb_ref[...].astype(bf)
    hl = hl_ref[...].astype(bf)
    hr = hr_ref[...].astype(bf)

    interL = (jnp.dot(hb, wbl[...], preferred_element_type=f32) * jnp.dot(
        hl, wnl[...], preferred_element_type=f32))
    aL = jnp.maximum(
        jnp.dot(interL.astype(bf), w1l[...], preferred_element_type=f32)
        + b1l[...], 0.0)
    ml_ref[...] = jnp.dot(
        aL.astype(bf), w2l[...], preferred_element_type=f32) + b2l[...]

    interR = (jnp.dot(hb, wbr[...], preferred_element_type=f32) * jnp.dot(
        hr, wnr[...], preferred_element_type=f32))
    aR = jnp.maximum(
        jnp.dot(interR.astype(bf), w1r[...], preferred_element_type=f32)
        + b1r[...], 0.0)
    mr_ref[...] = jnp.dot(
        aR.astype(bf), w2r[...], preferred_element_type=f32) + b2r[...]

    sk_ref[...] = (
        jnp.dot(hl, wnlt[...], preferred_element_type=f32)
        + jnp.dot(hr, wnrt[...], preferred_element_type=f32)
        + jnp.dot(hb, wst[...], preferred_element_type=f32)
        + bsk[...]
    ).astype(bf)


def _final_body(po_ref, ml_ref, mr_ref, sk_ref, ln_g, ln_b, wot, bo, out_ref):
    del po_ref
    x = ml_ref[...] + mr_ref[...] + sk_ref[...].astype(jnp.float32)
    m = jnp.mean(x, axis=-1, keepdims=True)
    xc = x - m
    v = jnp.mean(xc * xc, axis=-1, keepdims=True)
    xn = xc * lax.rsqrt(v + 1e-5) * ln_g[...] + ln_b[...]
    out_ref[...] = (
        jnp.dot(jnp.maximum(xn, 0.0), wot[...],
                preferred_element_type=jnp.float32)
        + bo[...]
    )


def _slice_spec(h):
    return pl.BlockSpec((TILE, BD), lambda i: (i + h * GRIDH, 0))


def _local_spec():
    return pl.BlockSpec((TILE, BD), lambda i: (i, 0))


def _w_spec(r, c):
    return pl.BlockSpec((r, c), lambda i: (0, 0))


def kernel(h_bond, bond_index, h_node, fL_Wb, fL_Wn, fL_W1, fL_b1, fL_W2,
           fL_b2, fR_Wb, fR_Wn, fR_W1, fR_b1, fR_W2, fR_b2, Wnl, bnl, Wnr,
           bnr, Ws, bs, ln_g, ln_b, Wo, bo):
    left = bond_index[0]
    right = bond_index[1]
    bf = jnp.bfloat16
    f32 = jnp.float32

    bsk = (bnl + bnr + bs).reshape(1, BD)
    ffn_w = (
        fL_Wb.T.astype(bf), fL_Wn.T.astype(bf),
        fL_W1.T.astype(bf), fL_b1.reshape(1, ID),
        fL_W2.T.astype(bf), fL_b2.reshape(1, BD),
        fR_Wb.T.astype(bf), fR_Wn.T.astype(bf),
        fR_W1.T.astype(bf), fR_b1.reshape(1, ID),
        fR_W2.T.astype(bf), fR_b2.reshape(1, BD),
        Wnl.T.astype(bf), Wnr.T.astype(bf), Ws.T.astype(bf), bsk,
    )
    ffn_w_specs = [
        _w_spec(BD, ID), _w_spec(ND, ID), _w_spec(ID, ID), _w_spec(1, ID),
        _w_spec(ID, BD), _w_spec(1, BD),
        _w_spec(BD, ID), _w_spec(ND, ID), _w_spec(ID, ID), _w_spec(1, ID),
        _w_spec(ID, BD), _w_spec(1, BD),
        _w_spec(ND, BD), _w_spec(ND, BD), _w_spec(BD, BD), _w_spec(1, BD),
    ]
    zeros_n = jnp.zeros((N, BD), f32)

    mLs, mRs, skips, sLs, sRs = [], [], [], [], []
    for h in range(NSPLIT):
        hnL, hnR = _gather_pair(h_node, left, h_node, right, h)
        mL, mR, skip = pl.pallas_call(
            _ffn_body,
            grid=(GRIDH,),
            in_specs=[_slice_spec(h), _local_spec(), _local_spec()]
            + ffn_w_specs,
            out_specs=[_local_spec(), _local_spec(), _local_spec()],
            out_shape=[
                jax.ShapeDtypeStruct((EH, BD), f32),
                jax.ShapeDtypeStruct((EH, BD), f32),
                jax.ShapeDtypeStruct((EH, BD), bf),
            ],
        )(h_bond, hnL, hnR, *ffn_w)
        sL_h, sR_h = _scatter_pair(mL, right, mR, left, zeros_n, h)
        mLs.append(mL)
        mRs.append(mR)
        skips.append(skip)
        sLs.append(sL_h)
        sRs.append(sR_h)

    sL = sLs[0]
    sR = sRs[0]
    for h in range(1, NSPLIT):
        sL = sL + sLs[h]
        sR = sR + sRs[h]

    out = None
    fin_w = (ln_g.reshape(1, BD), ln_b.reshape(1, BD), Wo.T,
             bo.reshape(1, BD))
    fin_w_specs = [_w_spec(1, BD), _w_spec(1, BD), _w_spec(BD, BD),
                   _w_spec(1, BD)]
    for h in range(NSPLIT):
        mLg, mRg = _gather_pair(sL, left, sR, right, h)
        if out is None:
            prev = jnp.zeros((8, BD), f32)
            aliases = {}
        else:
            prev = out
            aliases = {0: 0}
        out = pl.pallas_call(
            _final_body,
            grid=(GRIDH,),
            in_specs=[pl.BlockSpec(memory_space=pltpu.MemorySpace.HBM),
                      _local_spec(), _local_spec(), _local_spec()]
            + fin_w_specs,
            out_specs=pl.BlockSpec((TILE, BD), lambda i, h=h: (i + h * GRIDH,
                                                               0)),
            out_shape=jax.ShapeDtypeStruct((E, BD), f32),
            input_output_aliases=aliases,
        )(prev, mLg, mRg, skips[h], *fin_w)
    return out


# speedup vs baseline: 1.4728x; 1.0239x over previous
import functools

import jax
import jax.numpy as jnp
from jax import lax
from jax.experimental import pallas as pl
from jax.experimental.pallas import tpu as pltpu
from jax.experimental.pallas import tpu_sc as plsc

N = 10000
E = 320000
BD = 128
ND = 128
ID = 256

NC = 2
NS = 16
NW = NC * NS

CHUNK = 128
NCHUNK = E // CHUNK
ROWS_A = 624
ROWS_LAST = N - ROWS_A * (NS - 1)

NSPLIT = 4
EH = E // NSPLIT
HCHUNK = NCHUNK // NSPLIT

TILE = 4000
GRIDH = EH // TILE


@functools.lru_cache(maxsize=None)
def _sc_mesh():
    return plsc.VectorSubcoreMesh(
        core_axis_name="c", subcore_axis_name="s",
        num_cores=NC, num_subcores=NS)


def _gather_pair(tabA, idxA, tabB, idxB, h):
    niter = (HCHUNK + NW - 1) // NW

    @functools.partial(
        pl.kernel,
        out_type=(
            jax.ShapeDtypeStruct((EH, BD), jnp.float32),
            jax.ShapeDtypeStruct((EH, BD), jnp.float32),
        ),
        mesh=_sc_mesh(),
        scratch_types=(
            pltpu.VMEM((2, CHUNK), jnp.int32),
            pltpu.VMEM((2, CHUNK), jnp.int32),
            pltpu.VMEM((2, CHUNK, BD), jnp.float32),
            pltpu.VMEM((2, CHUNK, BD), jnp.float32),
            pltpu.SemaphoreType.DMA,
            pltpu.SemaphoreType.DMA,
            pltpu.SemaphoreType.DMA,
        ),
    )
    def k(tA, iA, tB, iB, oA, oB, ia_v, ib_v, ra_v, rb_v, si, sA, sB):
        wid = lax.axis_index("s") * NC + lax.axis_index("c")
        g0 = (h * HCHUNK + wid) * CHUNK

        cp0 = pltpu.async_copy(iA.at[pl.ds(g0, CHUNK)], ia_v.at[0], si)
        cp1 = pltpu.async_copy(iB.at[pl.ds(g0, CHUNK)], ib_v.at[0], si)
        cp0.wait()
        cp1.wait()

        def body(j, carry):
            lc = wid + j * NW
            gbase = (h * HCHUNK + lc) * CHUNK
            obase = lc * CHUNK
            slot = lax.rem(j, 2)
            nslot = 1 - slot

            @pl.when(lc + NW < HCHUNK)
            def _():
                nbase = gbase + NW * CHUNK
                pltpu.async_copy(iA.at[pl.ds(nbase, CHUNK)], ia_v.at[nslot], si)
                pltpu.async_copy(iB.at[pl.ds(nbase, CHUNK)], ib_v.at[nslot], si)

            @pl.when(lc < HCHUNK)
            def _():
                cpA = pltpu.async_copy(tA.at[ia_v.at[slot]], ra_v.at[slot], sA)
                cpB = pltpu.async_copy(tB.at[ib_v.at[slot]], rb_v.at[slot], sB)
                cpA.wait()
                cpB.wait()
                pltpu.sync_copy(ra_v.at[slot], oA.at[pl.ds(obase, CHUNK)])
                pltpu.sync_copy(rb_v.at[slot], oB.at[pl.ds(obase, CHUNK)])

            @pl.when(lc + NW < HCHUNK)
            def _():
                pltpu.make_async_copy(
                    iA.at[pl.ds(0, CHUNK)], ia_v.at[nslot], si).wait()
                pltpu.make_async_copy(
                    iB.at[pl.ds(0, CHUNK)], ib_v.at[nslot], si).wait()

            return carry

        lax.fori_loop(0, niter, body, 0)

    return k(tabA, idxA, tabB, idxB)


def _scatter_pair(mL, idx_r, mR, idx_l, zeros_n, h):
    niter = (HCHUNK + NS - 1) // NS

    @functools.partial(
        pl.kernel,
        out_type=(
            jax.ShapeDtypeStruct((N, BD), jnp.float32),
            jax.ShapeDtypeStruct((N, BD), jnp.float32),
        ),
        mesh=_sc_mesh(),
        scratch_types=(
            pltpu.VMEM((2, CHUNK), jnp.int32),
            pltpu.VMEM((2, CHUNK, BD), jnp.float32),
            pltpu.VMEM_SHARED((N, BD), jnp.float32),
            pltpu.SemaphoreType.DMA,
        ),
    )
    def k(mLr, iR, mRr, iL, zr, oL, oR, idx_v, rows_v, acc, sin):
        cid = lax.axis_index("c")
        sid = lax.axis_index("s")
        r0 = pl.multiple_of(sid * ROWS_A, 8)

        @pl.when(sid < NS - 1)
        def _():
            pltpu.sync_copy(zr.at[pl.ds(r0, ROWS_A)], acc.at[pl.ds(r0, ROWS_A)])

        @pl.when(sid == NS - 1)
        def _():
            pltpu.sync_copy(
                zr.at[pl.ds(r0, ROWS_LAST)], acc.at[pl.ds(r0, ROWS_LAST)]
            )

        plsc.subcore_barrier()

        lb0 = sid * CHUNK
        gb0 = h * HCHUNK * CHUNK + lb0

        @pl.when(cid == 0)
        def _():
            pltpu.sync_copy(iR.at[pl.ds(gb0, CHUNK)], idx_v.at[0])
            pltpu.sync_copy(mLr.at[pl.ds(lb0, CHUNK)], rows_v.at[0])

        @pl.when(cid == 1)
        def _():
            pltpu.sync_copy(iL.at[pl.ds(gb0, CHUNK)], idx_v.at[0])
            pltpu.sync_copy(mRr.at[pl.ds(lb0, CHUNK)], rows_v.at[0])

        def body(j, carry):
            lc = sid + j * NS
            slot = lax.rem(j, 2)
            nslot = 1 - slot
            nlbase = (lc + NS) * CHUNK
            ngbase = h * HCHUNK * CHUNK + nlbase

            @pl.when(jnp.logical_and(cid == 0, lc + NS < HCHUNK))
            def _():
                pltpu.async_copy(iR.at[pl.ds(ngbase, CHUNK)],
                                 idx_v.at[nslot], sin)
                pltpu.async_copy(mLr.at[pl.ds(nlbase, CHUNK)],
                                 rows_v.at[nslot], sin)

            @pl.when(jnp.logical_and(cid == 1, lc + NS < HCHUNK))
            def _():
                pltpu.async_copy(iL.at[pl.ds(ngbase, CHUNK)],
                                 idx_v.at[nslot], sin)
                pltpu.async_copy(mRr.at[pl.ds(nlbase, CHUNK)],
                                 rows_v.at[nslot], sin)

            @pl.when(lc < HCHUNK)
            def _():
                pltpu.sync_copy(rows_v.at[slot], acc.at[idx_v.at[slot]],
                                add=True)

            @pl.when(lc + NS < HCHUNK)
            def _():
                pltpu.make_async_copy(
                    iR.at[pl.ds(0, CHUNK)], idx_v.at[nslot], sin).wait()
                pltpu.make_async_copy(
                    mLr.at[pl.ds(0, CHUNK)], rows_v.at[nslot], sin).wait()

            return carry

        lax.fori_loop(0, niter, body, 0)
        plsc.subcore_barrier()

        @pl.when(jnp.logical_and(cid == 0, sid < NS - 1))
        def _():
            pltpu.sync_copy(acc.at[pl.ds(r0, ROWS_A)], oL.at[pl.ds(r0, ROWS_A)])

        @pl.when(jnp.logical_and(cid == 0, sid == NS - 1))
        def _():
            pltpu.sync_copy(
                acc.at[pl.ds(r0, ROWS_LAST)], oL.at[pl.ds(r0, ROWS_LAST)]
            )

        @pl.when(jnp.logical_and(cid == 1, sid < NS - 1))
        def _():
            pltpu.sync_copy(acc.at[pl.ds(r0, ROWS_A)], oR.at[pl.ds(r0, ROWS_A)])

        @pl.when(jnp.logical_and(cid == 1, sid == NS - 1))
        def _():
            pltpu.sync_copy(
                acc.at[pl.ds(r0, ROWS_LAST)], oR.at[pl.ds(r0, ROWS_LAST)]
            )

    return k(mL, idx_r, mR, idx_l, zeros_n)


def _ffn_body(hb_ref, hl_ref, hr_ref, wbl, wnl, w1l, b1l, w2l, b2l, wbr, wnr,
              w1r, b1r, w2r, b2r, wnlt, wnrt, wst, bsk, ml_ref, mr_ref,
              sk_ref):
    f32 = jnp.float32
    bf = jnp.bfloat16
    hb = hb_ref[...].astype(bf)
    hl = hl_ref[...].astype(bf)
    hr = hr_ref[...].astype(bf)

    interL = (jnp.dot(hb, wbl[...], preferred_element_type=f32) * jnp.dot(
        hl, wnl[...], preferred_element_type=f32))
    aL = jnp.maximum(
        jnp.dot(interL.astype(bf), w1l[...], preferred_element_type=f32)
        + b1l[...], 0.0)
    ml_ref[...] = jnp.dot(
        aL.astype(bf), w2l[...], preferred_element_type=f32) + b2l[...]

    interR = (jnp.dot(hb, wbr[...], preferred_element_type=f32) * jnp.dot(
        hr, wnr[...], preferred_element_type=f32))
    aR = jnp.maximum(
        jnp.dot(interR.astype(bf), w1r[...], preferred_element_type=f32)
        + b1r[...], 0.0)
    mr_ref[...] = jnp.dot(
        aR.astype(bf), w2r[...], preferred_element_type=f32) + b2r[...]

    sk_ref[...] = (
        jnp.dot(hl, wnlt[...], preferred_element_type=f32)
        + jnp.dot(hr, wnrt[...], preferred_element_type=f32)
        + jnp.dot(hb, wst[...], preferred_element_type=f32)
        + bsk[...]
    ).astype(bf)


def _final_body(po_ref, ml_ref, mr_ref, sk_ref, ln_g, ln_b, wot, bo, out_ref):
    del po_ref
    x = ml_ref[...] + mr_ref[...] + sk_ref[...].astype(jnp.float32)
    m = jnp.mean(x, axis=-1, keepdims=True)
    xc = x - m
    v = jnp.mean(xc * xc, axis=-1, keepdims=True)
    xn = xc * lax.rsqrt(v + 1e-5) * ln_g[...] + ln_b[...]
    out_ref[...] = (
        jnp.dot(jnp.maximum(xn, 0.0), wot[...],
                preferred_element_type=jnp.float32)
        + bo[...]
    )


def _slice_spec(h):
    return pl.BlockSpec((TILE, BD), lambda i: (i + h * GRIDH, 0))


def _local_spec():
    return pl.BlockSpec((TILE, BD), lambda i: (i, 0))


def _w_spec(r, c):
    return pl.BlockSpec((r, c), lambda i: (0, 0))


def kernel(h_bond, bond_index, h_node, fL_Wb, fL_Wn, fL_W1, fL_b1, fL_W2,
           fL_b2, fR_Wb, fR_Wn, fR_W1, fR_b1, fR_W2, fR_b2, Wnl, bnl, Wnr,
           bnr, Ws, bs, ln_g, ln_b, Wo, bo):
    left = bond_index[0]
    right = bond_index[1]
    bf = jnp.bfloat16
    f32 = jnp.float32

    bsk = (bnl + bnr + bs).reshape(1, BD)
    ffn_w = (
        fL_Wb.T.astype(bf), fL_Wn.T.astype(bf),
        fL_W1.T.astype(bf), fL_b1.reshape(1, ID),
        fL_W2.T.astype(bf), fL_b2.reshape(1, BD),
        fR_Wb.T.astype(bf), fR_Wn.T.astype(bf),
        fR_W1.T.astype(bf), fR_b1.reshape(1, ID),
        fR_W2.T.astype(bf), fR_b2.reshape(1, BD),
        Wnl.T.astype(bf), Wnr.T.astype(bf), Ws.T.astype(bf), bsk,
    )
    ffn_w_specs = [
        _w_spec(BD, ID), _w_spec(ND, ID), _w_spec(ID, ID), _w_spec(1, ID),
        _w_spec(ID, BD), _w_spec(1, BD),
        _w_spec(BD, ID), _w_spec(ND, ID), _w_spec(ID, ID), _w_spec(1, ID),
        _w_spec(ID, BD), _w_spec(1, BD),
        _w_spec(ND, BD), _w_spec(ND, BD), _w_spec(BD, BD), _w_spec(1, BD),
    ]
    zeros_n = jnp.zeros((N, BD), f32)

    mLs, mRs, skips, sLs, sRs = [], [], [], [], []
    for h in range(NSPLIT):
        hnL, hnR = _gather_pair(h_node, left, h_node, right, h)
        mL, mR, skip = pl.pallas_call(
            _ffn_body,
            grid=(GRIDH,),
            in_specs=[_slice_spec(h), _local_spec(), _local_spec()]
            + ffn_w_specs,
            out_specs=[_local_spec(), _local_spec(), _local_spec()],
            out_shape=[
                jax.ShapeDtypeStruct((EH, BD), f32),
                jax.ShapeDtypeStruct((EH, BD), f32),
                jax.ShapeDtypeStruct((EH, BD), bf),
            ],
        )(h_bond, hnL, hnR, *ffn_w)
        sL_h, sR_h = _scatter_pair(mL, right, mR, left, zeros_n, h)
        mLs.append(mL)
        mRs.append(mR)
        skips.append(skip)
        sLs.append(sL_h)
        sRs.append(sR_h)

    sL = sLs[0]
    sR = sRs[0]
    for h in range(1, NSPLIT):
        sL = sL + sLs[h]
        sR = sR + sRs[h]

    out = None
    fin_w = (ln_g.reshape(1, BD), ln_b.reshape(1, BD), Wo.T,
             bo.reshape(1, BD))
    fin_w_specs = [_w_spec(1, BD), _w_spec(1, BD), _w_spec(BD, BD),
                   _w_spec(1, BD)]
    for h in range(NSPLIT):
        mLg, mRg = _gather_pair(sL, left, sR, right, h)
        if out is None:
            prev = jnp.zeros((8, BD), f32)
            aliases = {}
        else:
            prev = out
            aliases = {0: 0}
        out = pl.pallas_call(
            _final_body,
            grid=(GRIDH,),
            in_specs=[pl.BlockSpec(memory_space=pltpu.MemorySpace.HBM),
                      _local_spec(), _local_spec(), _local_spec()]
            + fin_w_specs,
            out_specs=pl.BlockSpec((TILE, BD), lambda i, h=h: (i + h * GRIDH,
                                                               0)),
            out_shape=jax.ShapeDtypeStruct((E, BD), f32),
            input_output_aliases=aliases,
        )(prev, mLg, mRg, skips[h], *fin_w)
    return out
